# Initial kernel scaffold; baseline (speedup 1.0000x reference)
import jax
import jax.numpy as jnp
from jax.experimental import pallas as pl


def kernel(frames):
    raise NotImplementedError("write your pallas kernel here")


# speedup vs baseline: 1.1962x; 1.1962x over previous
import jax
import jax.numpy as jnp
from jax.experimental import pallas as pl
from jax.experimental.pallas import tpu as pltpu

_ALPHA = 4


def _pack_body(jmap_ref, in_ref, slow_ref, fast_ref):
    fast_ref[...] = in_ref[...]
    slow_ref[...] = in_ref[...]


def kernel(frames):
    C, T, H, W = frames.shape
    n_slow = T // _ALPHA
    idx = jnp.linspace(0.0, T - 1, n_slow).astype(jnp.int32)
    jmap = jnp.searchsorted(idx, jnp.arange(T, dtype=jnp.int32)).astype(jnp.int32)

    grid_spec = pltpu.PrefetchScalarGridSpec(
        num_scalar_prefetch=1,
        grid=(T,),
        in_specs=[
            pl.BlockSpec((C, 1, H, W), lambda t, jm: (0, t, 0, 0)),
        ],
        out_specs=[
            pl.BlockSpec((C, 1, H, W), lambda t, jm: (0, jm[t], 0, 0)),
            pl.BlockSpec((C, 1, H, W), lambda t, jm: (0, t, 0, 0)),
        ],
    )
    slow, fast = pl.pallas_call(
        _pack_body,
        grid_spec=grid_spec,
        out_shape=(
            jax.ShapeDtypeStruct((C, n_slow, H, W), frames.dtype),
            jax.ShapeDtypeStruct((C, T, H, W), frames.dtype),
        ),
    )(jmap, frames)
    return (slow, fast)
